# baseline (speedup 1.0000x reference)
import functools

import jax
import jax.numpy as jnp
from jax import lax
from jax.experimental import pallas as pl
from jax.experimental.pallas import tpu as pltpu
from jax.experimental.pallas import tpu_sc as plsc


@functools.lru_cache(maxsize=None)
def _build_expand(SL: int, D: int):
    info = plsc.get_sparse_core_info()
    NC, NS = info.num_cores, info.num_subcores
    NW = NC * NS
    assert SL % NW == 0 and SL % 128 == 0 and D % 8 == 0
    RPW = SL // NW
    DT = D // 8
    JT = SL // 128
    PLANE_ROWS = DT * JT * 8

    mesh = plsc.VectorSubcoreMesh(core_axis_name="c", subcore_axis_name="s")

    @functools.partial(
        pl.kernel,
        mesh=mesh,
        out_type=jax.ShapeDtypeStruct((SL * SL * D // 128, 128), jnp.float32),
        scratch_types=[
            pltpu.VMEM_SHARED((8, D, 2 * SL), jnp.float32),
            pltpu.SemaphoreType.DMA,
        ],
        compiler_params=pltpu.CompilerParams(use_tc_tiling_on_sc=False),
    )
    def expand(wtf_hbm, out_hbm, spm, ssem):
        cid = lax.axis_index("c")
        sid = lax.axis_index("s")
        wid = sid * NC + cid
        base = wid * RPW
        @pl.when(sid == 0)
        def _stage():
            pltpu.sync_copy(wtf_hbm, spm)
        plsc.subcore_barrier()

        def body(t, _):
            i = base + t
            c0 = SL - 1 - i
            ph = lax.rem(c0, 8)
            a0 = pl.multiple_of(c0 - ph, 8)
            @pl.when(t > 0)
            def _drain():
                for _ in range(DT * JT):
                    pltpu.make_async_copy(
                        spm.at[0, pl.ds(0, 8), pl.ds(0, 128)],
                        out_hbm.at[pl.ds(0, 8)],
                        ssem,
                    ).wait()
            row0 = pl.multiple_of(i * PLANE_ROWS, 8)
            for dt in range(DT):
                for jt in range(JT):
                    pltpu.async_copy(
                        spm.at[ph, pl.ds(8 * dt, 8),
                               pl.ds(a0 + 128 * jt, 128)],
                        out_hbm.at[pl.ds(row0 + (dt * JT + jt) * 8, 8)],
                        ssem,
                    )
            return _

        lax.fori_loop(0, RPW, body, None)
        for _ in range(DT * JT):
            pltpu.make_async_copy(
                spm.at[0, pl.ds(0, 8), pl.ds(0, 128)],
                out_hbm.at[pl.ds(0, 8)],
                ssem,
            ).wait()

    return expand


def kernel(seq_len, relative_positions_weight):
    V, D = relative_positions_weight.shape
    SL = (V + 1) // 2
    wtf = jnp.flip(relative_positions_weight, axis=0).T
    wtf = jnp.concatenate([wtf, jnp.zeros((D, 9), wtf.dtype)], axis=1)
    wtf8 = jnp.stack([wtf[:, p:p + 2 * SL] for p in range(8)])
    out2d = _build_expand(SL, D)(wtf8)
    out5 = out2d.reshape(SL, D // 8, SL // 128, 8, 128)
    return out5.transpose(0, 2, 4, 1, 3).reshape(SL, SL, D)


# speedup vs baseline: 1.0268x; 1.0268x over previous
import functools

import jax
import jax.numpy as jnp
from jax import lax
from jax.experimental import pallas as pl
from jax.experimental.pallas import tpu as pltpu
from jax.experimental.pallas import tpu_sc as plsc


@functools.lru_cache(maxsize=None)
def _build_expand(SL: int, D: int):
    info = plsc.get_sparse_core_info()
    NC, NS = info.num_cores, info.num_subcores
    NW = NC * NS
    assert SL % NW == 0 and SL % 128 == 0 and D % 16 == 0
    RPW = SL // NW
    DT = D // 8
    JT = SL // 128
    HT = DT // 2

    mesh = plsc.VectorSubcoreMesh(core_axis_name="c", subcore_axis_name="s")

    @functools.partial(
        pl.kernel,
        mesh=mesh,
        out_type=jax.ShapeDtypeStruct((SL * DT, JT * 8, 128), jnp.float32),
        scratch_types=[
            pltpu.VMEM_SHARED((8, DT, 8, 2 * SL), jnp.float32),
            pltpu.VMEM((HT, JT * 8, 128), jnp.float32),
            pltpu.VMEM((HT, JT * 8, 128), jnp.float32),
            pltpu.SemaphoreType.DMA,
            pltpu.SemaphoreType.DMA,
            pltpu.SemaphoreType.DMA,
        ],
        compiler_params=pltpu.CompilerParams(use_tc_tiling_on_sc=False),
    )
    def expand(wtf_hbm, out_hbm, spm, buf0, buf1, gsem, ssem0, ssem1):
        cid = lax.axis_index("c")
        sid = lax.axis_index("s")
        wid = sid * NC + cid
        base = wid * RPW
        @pl.when(sid == 0)
        def _stage():
            pltpu.sync_copy(wtf_hbm, spm)
        plsc.subcore_barrier()

        bufs = (buf0, buf1)
        ssems = (ssem0, ssem1)

        def body(t, _):
            i = base + t
            c0 = SL - 1 - i
            ph = lax.rem(c0, 8)
            a0 = pl.multiple_of(c0 - ph, 8)
            for h in range(2):
                buf, ssem = bufs[h], ssems[h]
                dst = i * DT + h * HT
                @pl.when(t > 0)
                def _drain():
                    pltpu.make_async_copy(
                        buf, out_hbm.at[pl.ds(dst - DT, HT)], ssem
                    ).wait()
                gathers = [
                    pltpu.async_copy(
                        spm.at[ph, pl.ds(h * HT, HT), :,
                               pl.ds(a0 + 128 * jt, 128)],
                        buf.at[:, pl.ds(8 * jt, 8), :],
                        gsem,
                    )
                    for jt in range(JT)
                ]
                for g in gathers:
                    g.wait()
                pltpu.async_copy(buf, out_hbm.at[pl.ds(dst, HT)], ssem)
            return _

        lax.fori_loop(0, RPW, body, None)
        last = base + RPW - 1
        for h in range(2):
            pltpu.make_async_copy(
                bufs[h],
                out_hbm.at[pl.ds(last * DT + h * HT, HT)],
                ssems[h],
            ).wait()

    return expand


def kernel(seq_len, relative_positions_weight):
    V, D = relative_positions_weight.shape
    SL = (V + 1) // 2
    DT = D // 8
    wtf = jnp.flip(relative_positions_weight, axis=0).T
    wtf = jnp.concatenate([wtf, jnp.zeros((D, 9), wtf.dtype)], axis=1)
    wtf8 = jnp.stack([wtf[:, p:p + 2 * SL] for p in range(8)])
    wtf8 = wtf8.reshape(8, DT, 8, 2 * SL)
    out3d = _build_expand(SL, D)(wtf8)
    out5 = out3d.reshape(SL, DT, SL // 128, 8, 128)
    return out5.transpose(0, 2, 4, 1, 3).reshape(SL, SL, D)


# speedup vs baseline: 1.3700x; 1.3343x over previous
import functools

import jax
import jax.numpy as jnp
from jax import lax
from jax.experimental import pallas as pl
from jax.experimental.pallas import tpu as pltpu
from jax.experimental.pallas import tpu_sc as plsc


@functools.lru_cache(maxsize=None)
def _build_expand(SL: int, D: int):
    info = plsc.get_sparse_core_info()
    NC, NS = info.num_cores, info.num_subcores
    NW = NC * NS
    assert SL % (2 * NW) == 0 and SL % 128 == 0 and D % 16 == 0
    RPW = SL // NW
    DT = D // 8
    JT = SL // 128
    HT = DT // 2
    HROWS = HT * JT * 8
    PLANE_ROWS = 2 * HROWS

    mesh = plsc.VectorSubcoreMesh(core_axis_name="c", subcore_axis_name="s")

    @functools.partial(
        pl.kernel,
        mesh=mesh,
        out_type=jax.ShapeDtypeStruct((SL * SL * D // 128, 128), jnp.float32),
        scratch_types=[
            pltpu.VMEM_SHARED((8, D, 2 * SL), jnp.float32),
            pltpu.VMEM((HROWS, 128), jnp.float32),
            pltpu.VMEM((HROWS, 128), jnp.float32),
            pltpu.SemaphoreType.DMA,
            pltpu.SemaphoreType.DMA,
            pltpu.SemaphoreType.DMA,
            pltpu.SemaphoreType.DMA,
        ],
        compiler_params=pltpu.CompilerParams(use_tc_tiling_on_sc=False),
    )
    def expand(wtf_hbm, out_hbm, spm, buf0, buf1, gsem, ssem0, ssem1, dsem):
        cid = lax.axis_index("c")
        sid = lax.axis_index("s")
        wid = sid * NC + cid
        base = wid * RPW
        @pl.when(sid == 0)
        def _stage():
            pltpu.sync_copy(wtf_hbm, spm)
        plsc.subcore_barrier()

        bufs = (buf0, buf1)
        ssems = (ssem0, ssem1)

        def phase_of(i):
            c0 = SL - 1 - i
            ph = lax.rem(c0, 8)
            return ph, pl.multiple_of(c0 - ph, 8)

        def body(t, _):
            ib = base + 2 * t + 1
            phb, a0b = phase_of(ib)
            rowb = pl.multiple_of(ib * PLANE_ROWS, 8)
            @pl.when(t > 0)
            def _drain_direct():
                for _ in range(DT * JT):
                    pltpu.make_async_copy(
                        spm.at[0, pl.ds(0, 8), pl.ds(0, 128)],
                        out_hbm.at[pl.ds(0, 8)],
                        dsem,
                    ).wait()
            for dt in range(DT):
                for jt in range(JT):
                    pltpu.async_copy(
                        spm.at[phb, pl.ds(8 * dt, 8),
                               pl.ds(a0b + 128 * jt, 128)],
                        out_hbm.at[pl.ds(rowb + (dt * JT + jt) * 8, 8)],
                        dsem,
                    )
            ia = base + 2 * t
            pha, a0a = phase_of(ia)
            for h in range(2):
                buf, ssem = bufs[h], ssems[h]
                dst_row = ia * PLANE_ROWS + h * HROWS
                @pl.when(t > 0)
                def _drain_bounce():
                    pltpu.make_async_copy(
                        buf,
                        out_hbm.at[pl.ds(dst_row - 2 * PLANE_ROWS, HROWS)],
                        ssem,
                    ).wait()
                gathers = []
                for dtl in range(HT):
                    dt = h * HT + dtl
                    for jt in range(JT):
                        gathers.append(
                            pltpu.async_copy(
                                spm.at[pha, pl.ds(8 * dt, 8),
                                       pl.ds(a0a + 128 * jt, 128)],
                                buf.at[pl.ds((dtl * JT + jt) * 8, 8)],
                                gsem,
                            )
                        )
                for g in gathers:
                    g.wait()
                pltpu.async_copy(
                    buf, out_hbm.at[pl.ds(dst_row, HROWS)], ssem
                )
            return _

        lax.fori_loop(0, RPW // 2, body, None)
        for _ in range(DT * JT):
            pltpu.make_async_copy(
                spm.at[0, pl.ds(0, 8), pl.ds(0, 128)],
                out_hbm.at[pl.ds(0, 8)],
                dsem,
            ).wait()
        last_a = base + RPW - 2
        for h in range(2):
            pltpu.make_async_copy(
                bufs[h],
                out_hbm.at[pl.ds(last_a * PLANE_ROWS + h * HROWS, HROWS)],
                ssems[h],
            ).wait()

    return expand


def kernel(seq_len, relative_positions_weight):
    V, D = relative_positions_weight.shape
    SL = (V + 1) // 2
    wtf = jnp.flip(relative_positions_weight, axis=0).T
    wtf = jnp.concatenate([wtf, jnp.zeros((D, 9), wtf.dtype)], axis=1)
    wtf8 = jnp.stack([wtf[:, p:p + 2 * SL] for p in range(8)])
    out2d = _build_expand(SL, D)(wtf8)
    out5 = out2d.reshape(SL, D // 8, SL // 128, 8, 128)
    return out5.transpose(0, 2, 4, 1, 3).reshape(SL, SL, D)
